# baseline (speedup 1.0000x reference)
import functools

import jax
import jax.numpy as jnp
from jax import lax
from jax.experimental import pallas as pl
from jax.experimental.pallas import tpu as pltpu
from jax.experimental.pallas import tpu_sc as plsc

D = 64
NC = 2
NS = 16
NW = NC * NS
H = 8192
ICHUNK = 8192
NSLOT = 8
MAXR = 8


def _sc_sweep_gather(emb_t, idx):
    _, V = emb_t.shape
    N = idx.shape[0]
    ncols_tot = (V + 127) // 128
    q, rem = divmod(ncols_tot, NW)
    n_out = N + 16 * NW
    nichunk = N // ICHUNK
    mesh = plsc.VectorSubcoreMesh(core_axis_name="c", subcore_axis_name="s")

    @functools.partial(
        pl.kernel,
        out_type=jax.ShapeDtypeStruct((n_out, 128), jnp.float32),
        mesh=mesh,
        scratch_types=[
            pltpu.VMEM((ICHUNK,), jnp.int32),
            pltpu.VMEM((H + 16,), jnp.int32),
            pltpu.VMEM((H + 16,), jnp.int32),
            pltpu.VMEM((2, D, 128), jnp.float32),
            pltpu.VMEM((16,), jnp.int32),
            pltpu.VMEM((16,), jnp.int32),
            pltpu.VMEM((NSLOT, 16, 128), jnp.float32),
            pltpu.VMEM((NSLOT, 16), jnp.int32),
            pltpu.SMEM((4,), jnp.int32),
            pltpu.SemaphoreType.DMA,
            pltpu.SemaphoreType.DMA,
            pltpu.SemaphoreType.DMA,
        ],
        compiler_params=pltpu.CompilerParams(needs_layout_passes=False),
    )
    def k(emb_hbm, idx_hbm, out_hbm, ichunk, midx, mpos, slabs,
          acti, actp, stage, posb, sref, sl_e, sl_o, sc_sem):
        wid = lax.axis_index("s") * NC + lax.axis_index("c")
        c0 = wid * q + jnp.minimum(wid, rem)
        ncols = q + jnp.where(wid < rem, 1, 0)
        lo = c0 * 128
        hi = (c0 + ncols) * 128
        dummy = N + wid * 16
        iota = lax.iota(jnp.int32, 16)

        def bin_round(r):
            sref[0] = 0
            sref[1] = 0

            @pl.loop(0, nichunk)
            def _(cc):
                pltpu.sync_copy(idx_hbm.at[pl.ds(cc * ICHUNK, ICHUNK)],
                                ichunk)

                @pl.loop(0, ICHUNK // 16)
                def _(vv):
                    cnt = sref[0]
                    seen = sref[1]
                    x = ichunk[pl.ds(vv * 16, 16)]
                    m = (x >= lo) & (x < hi)
                    mi = jnp.where(m, 1, 0)
                    rank = seen + jnp.cumsum(mi)
                    keep = m & (rank > r * H) & (rank <= r * H + H)
                    plsc.store_compressed(midx.at[pl.ds(cnt, 16)], x, mask=keep)
                    pos = cc * ICHUNK + vv * 16 + iota
                    plsc.store_compressed(mpos.at[pl.ds(cnt, 16)], pos, mask=keep)
                    sref[0] = cnt + plsc.all_reduce_population_count(keep)[0]
                    sref[1] = seen + plsc.all_reduce_population_count(m)[0]

        def fire_slab(c, par, sem):
            pltpu.async_copy(emb_hbm.at[:, pl.ds((c0 + c) * 128, 128)],
                             slabs.at[par], sem)

        def wait_slab(par, sem):
            pltpu.make_async_copy(emb_hbm.at[:, pl.ds(0, 128)],
                                  slabs.at[par], sem).wait()

        def wait_scat():
            pltpu.make_async_copy(stage.at[0],
                                  out_hbm.at[pl.ds(0, 16)], sc_sem).wait()

        def process_slab(c, par):
            c_abs = c0 + c
            cnt = sref[0]

            @pl.loop(0, (cnt + 15) // 16)
            def _(vv):
                x = midx[pl.ds(vv * 16, 16)]
                valid = (vv * 16 + iota) < cnt
                m = valid & ((x >> 7) == c_abs)
                mcnt = plsc.all_reduce_population_count(m)[0]

                @pl.when(mcnt > 0)
                def _():
                    p = mpos[pl.ds(vv * 16, 16)]
                    gi = sref[2]

                    @pl.when(gi >= NSLOT)
                    def _():
                        wait_scat()
                    slot = gi % NSLOT
                    plsc.store_compressed(acti.at[pl.ds(0, 16)], x, mask=m)
                    plsc.store_compressed(actp.at[pl.ds(0, 16)], p, mask=m)
                    av = acti[pl.ds(0, 16)]
                    ap = actp[pl.ds(0, 16)]
                    pp = jnp.where(iota < mcnt, ap, dummy + iota)
                    posb[slot, pl.ds(0, 16)] = pp
                    for kk in range(16):
                        col = jnp.full((16,), av[kk] & 127, jnp.int32)
                        pv = jnp.full((16,), par, jnp.int32)
                        for jj in range(D // 16):
                            g = plsc.load_gather(
                                slabs, [pv, iota + 16 * jj, col])
                            stage[slot, kk, pl.ds(16 * jj, 16)] = g
                    pltpu.async_copy(stage.at[slot],
                                     out_hbm.at[posb.at[slot]], sc_sem)
                    sref[2] = gi + 1

        def sweep():
            fire_slab(0, 0, sl_e)

            @pl.loop(0, (ncols + 1) // 2)
            def _(pr):
                c = 2 * pr

                @pl.when(c + 1 < ncols)
                def _():
                    fire_slab(c + 1, 1, sl_o)
                wait_slab(0, sl_e)
                process_slab(c, 0)

                @pl.when(c + 2 < ncols)
                def _():
                    fire_slab(c + 2, 0, sl_e)

                @pl.when(c + 1 < ncols)
                def _():
                    wait_slab(1, sl_o)
                    process_slab(c + 1, 1)

        sref[2] = 0

        @pl.loop(0, MAXR)
        def _(r):
            @pl.when((r == 0) | (r * H < sref[1]))
            def _():
                bin_round(r)
                sweep()

        gi_end = sref[2]

        @pl.loop(0, jnp.minimum(gi_end, NSLOT))
        def _(_i):
            wait_scat()

    return k(emb_t, idx)


def _tc_mlp(x, W1, b1, W2, b2):
    n = 65536
    d = D
    blk = 4096

    def body(x_ref, w1_ref, b1_ref, w2_ref, b2_ref, o_ref):
        xs = x_ref[...][:, :d]
        h = jnp.maximum(
            jnp.dot(xs, w1_ref[...], preferred_element_type=jnp.float32)
            + b1_ref[...], 0.0)
        o_ref[...] = (
            jnp.dot(h, w2_ref[...], preferred_element_type=jnp.float32)
            + b2_ref[...])

    return pl.pallas_call(
        body,
        grid=(n // blk,),
        in_specs=[
            pl.BlockSpec((blk, 2 * d), lambda i: (i, 0)),
            pl.BlockSpec((d, d), lambda i: (0, 0)),
            pl.BlockSpec((1, d), lambda i: (0, 0)),
            pl.BlockSpec((d, d), lambda i: (0, 0)),
            pl.BlockSpec((1, d), lambda i: (0, 0)),
        ],
        out_specs=pl.BlockSpec((blk, d), lambda i: (i, 0)),
        out_shape=jax.ShapeDtypeStruct((n, d), jnp.float32),
    )(x, W1, b1.reshape(1, d), W2, b2.reshape(1, d))


def kernel(class_indices, embedding, W1, b1, W2, b2):
    if class_indices.ndim == 1:
        class_indices = class_indices[:, None]
    q_, b_ = class_indices.shape
    idx = class_indices.reshape(-1).astype(jnp.int32)
    gathered = _sc_sweep_gather(embedding.T, idx)
    out = _tc_mlp(gathered, W1, b1, W2, b2)
    return out.reshape(q_, b_, D)


# speedup vs baseline: 4.5653x; 4.5653x over previous
import functools

import jax
import jax.numpy as jnp
from jax import lax
from jax.experimental import pallas as pl
from jax.experimental.pallas import tpu as pltpu
from jax.experimental.pallas import tpu_sc as plsc

D = 64
NC = 2
NS = 16
NW = NC * NS
GROUP = 16
NGROUP = 8


def _sc_gather(emb, idx3):
    V, _ = emb.shape
    nw, R, C = idx3.shape
    per_w = R * C
    n = nw * per_w
    ngrp = per_w // GROUP
    mesh = plsc.VectorSubcoreMesh(core_axis_name="c", subcore_axis_name="s")

    @functools.partial(
        pl.kernel,
        out_type=jax.ShapeDtypeStruct((n, D), jnp.float32),
        mesh=mesh,
        scratch_types=[
            pltpu.VMEM((R, C), jnp.int32),
            pltpu.VMEM((2 * GROUP, 8, D), jnp.float32),
            pltpu.VMEM((2, GROUP, D), jnp.float32),
            pltpu.SemaphoreType.DMA,
            pltpu.SemaphoreType.DMA,
            pltpu.SemaphoreType.DMA,
            pltpu.SemaphoreType.DMA,
        ],
    )
    def k(emb_hbm, idx_hbm, out_hbm, idx_v, slabs, rows, s_e, s_o, o_e, o_o):
        wid = lax.axis_index("s") * NC + lax.axis_index("c")
        base = wid * per_w
        pltpu.sync_copy(idx_hbm.at[wid], idx_v)

        def load_grp(g):
            return idx_v[g // NGROUP, pl.ds((g % NGROUP) * GROUP, GROUP)]

        def fire(g, par, sem):
            v = load_grp(g)
            half = par * GROUP
            for nn in range(GROUP):
                start = pl.multiple_of(v[nn] & jnp.int32(-8), 8)
                pltpu.async_copy(
                    emb_hbm.at[pl.ds(start, 8)],
                    slabs.at[half + nn], sem)

        def drain_extract(g, par, sem):
            v = load_grp(g)
            half = par * GROUP
            for nn in range(GROUP):
                pltpu.make_async_copy(
                    emb_hbm.at[pl.ds(0, 8)],
                    slabs.at[half + nn], sem).wait()
            for nn in range(GROUP):
                r = v[nn] & 7
                for j in range(D // 16):
                    rows[par, nn, pl.ds(16 * j, 16)] = (
                        slabs[half + nn, r, pl.ds(16 * j, 16)])

        def write_out(g, par, sem):
            pltpu.async_copy(
                rows.at[par],
                out_hbm.at[pl.ds(base + g * GROUP, GROUP)], sem)

        def wait_out(par, sem):
            pltpu.make_async_copy(
                rows.at[par],
                out_hbm.at[pl.ds(base, GROUP)], sem).wait()

        fire(0, 0, s_e)

        @pl.loop(0, ngrp // 2)
        def _(p):
            g = 2 * p

            @pl.when(g >= 2)
            def _():
                wait_out(0, o_e)
            fire(g + 1, 1, s_o)
            drain_extract(g, 0, s_e)
            write_out(g, 0, o_e)

            @pl.when(g >= 1)
            def _():
                wait_out(1, o_o)

            @pl.when(g + 2 < ngrp)
            def _():
                fire(g + 2, 0, s_e)
            drain_extract(g + 1, 1, s_o)
            write_out(g + 1, 1, o_o)

        wait_out(0, o_e)
        wait_out(1, o_o)

    return k(emb, idx3)


def _tc_mlp(x, W1, b1, W2, b2):
    n, d = x.shape
    blk = 4096

    def body(x_ref, w1_ref, b1_ref, w2_ref, b2_ref, o_ref):
        h = jnp.maximum(
            jnp.dot(x_ref[...], w1_ref[...], preferred_element_type=jnp.float32)
            + b1_ref[...], 0.0)
        o_ref[...] = (
            jnp.dot(h, w2_ref[...], preferred_element_type=jnp.float32)
            + b2_ref[...])

    return pl.pallas_call(
        body,
        grid=(n // blk,),
        in_specs=[
            pl.BlockSpec((blk, d), lambda i: (i, 0)),
            pl.BlockSpec((d, d), lambda i: (0, 0)),
            pl.BlockSpec((1, d), lambda i: (0, 0)),
            pl.BlockSpec((d, d), lambda i: (0, 0)),
            pl.BlockSpec((1, d), lambda i: (0, 0)),
        ],
        out_specs=pl.BlockSpec((blk, d), lambda i: (i, 0)),
        out_shape=jax.ShapeDtypeStruct((n, d), jnp.float32),
    )(x, W1, b1.reshape(1, d), W2, b2.reshape(1, d))


def kernel(class_indices, embedding, W1, b1, W2, b2):
    if class_indices.ndim == 1:
        class_indices = class_indices[:, None]
    q, b = class_indices.shape
    n = q * b
    per_w = n // NW
    idx3 = class_indices.reshape(NW, per_w // 128, 128).astype(jnp.int32)
    gathered = _sc_gather(embedding, idx3)
    out = _tc_mlp(gathered, W1, b1, W2, b2)
    return out.reshape(q, b, D)
